# baseline (speedup 1.0000x reference)
import functools

import jax
import jax.numpy as jnp
from jax import lax
from jax.experimental import pallas as pl
from jax.experimental.pallas import tpu as pltpu
from jax.experimental.pallas import tpu_sc as plsc

N = 10000
E = 320000
ELG = 640000
HID = 64
FIN = 128
OUT = 40

NC = 2
NS = 16
NW = NC * NS
EH = E // NC
EHP = 160256
ND = 10240
CW = 32

_mesh = plsc.VectorSubcoreMesh(
    core_axis_name="c", subcore_axis_name="s", num_cores=NC, num_subcores=NS)
_sc_params = pltpu.CompilerParams(use_tc_tiling_on_sc=False)

f32 = jnp.float32
i32 = jnp.int32

SR = 320


def _fill_zero2d(stage):
    def zfill(i, _):
        for jj in range(CW // 16):
            stage[i, pl.ds(jj * 16, 16)] = jnp.zeros((16,), f32)
        return 0
    lax.fori_loop(0, SR, zfill, 0)


def _acc_zero_n(dacc, stage, s):
    @pl.when(s < NS - 1)
    def _():
        def cp(k, _):
            pltpu.sync_copy(stage, dacc.at[pl.ds(s * 640 + k * SR, SR)])
            return 0
        lax.fori_loop(0, 2, cp, 0)

    @pl.when(s == NS - 1)
    def _():
        pltpu.sync_copy(stage, dacc.at[pl.ds(9600, SR)])
        pltpu.sync_copy(stage.at[pl.ds(0, 80)], dacc.at[pl.ds(9920, 80)])


def _acc_readout_n(dacc, hbm_ref, stage, cbase, s):
    @pl.when(s < NS - 1)
    def _():
        def cp(k, _):
            r0 = s * 640 + k * SR
            pltpu.sync_copy(dacc.at[pl.ds(r0, SR)], stage)
            pltpu.sync_copy(stage, hbm_ref.at[pl.ds(cbase + r0, SR)])
            return 0
        lax.fori_loop(0, 2, cp, 0)

    @pl.when(s == NS - 1)
    def _():
        pltpu.sync_copy(dacc.at[pl.ds(9600, SR)], stage)
        pltpu.sync_copy(stage, hbm_ref.at[pl.ds(cbase + 9600, SR)])
        pltpu.sync_copy(dacc.at[pl.ds(9920, 80)], stage.at[pl.ds(0, 80)])
        pltpu.sync_copy(stage.at[pl.ds(0, 80)], hbm_ref.at[pl.ds(cbase + 9920, 80)])


BE = 2560

def _t1_body(et_ref, ea_ref, wa_ref, wb_ref, b_ref, wg_ref, av_ref,
             hga_ref, hgb_ref, sd_ref):
    z = et_ref[...] @ wa_ref[...] + ea_ref[...] @ wb_ref[...] + b_ref[...]
    h = jnp.maximum(z, 0.0)
    hg = h @ wg_ref[...]
    hga_ref[...] = hg[:, 0:CW]
    hgb_ref[...] = hg[:, CW:HID]
    s = lax.dot_general(av_ref[...], hg, (((1,), (1,)), ((), ())),
                        preferred_element_type=f32)
    sd_ref[0, :, :] = s


def _t1(et, ea, wa, wb, b, wg, av2):
    return pl.pallas_call(
        _t1_body,
        grid=(E // BE,),
        in_specs=[
            pl.BlockSpec((BE, 16), lambda i: (i, 0)),
            pl.BlockSpec((BE, 16), lambda i: (i, 0)),
            pl.BlockSpec((16, HID), lambda i: (0, 0)),
            pl.BlockSpec((16, HID), lambda i: (0, 0)),
            pl.BlockSpec((1, HID), lambda i: (0, 0)),
            pl.BlockSpec((HID, HID), lambda i: (0, 0)),
            pl.BlockSpec((2, HID), lambda i: (0, 0)),
        ],
        out_specs=[
            pl.BlockSpec((BE, CW), lambda i: (i, 0)),
            pl.BlockSpec((BE, CW), lambda i: (i, 0)),
            pl.BlockSpec((1, 2, BE), lambda i: (i, 0, 0)),
        ],
        out_shape=[
            jax.ShapeDtypeStruct((E, CW), f32),
            jax.ShapeDtypeStruct((E, CW), f32),
            jax.ShapeDtypeStruct((E // BE, 2, BE), f32),
        ],
    )(et, ea, wa, wb, b, wg, av2)


B1 = 4000
EPS1 = ELG // NS
DRS = EHP // NS


@functools.partial(
    pl.kernel,
    out_type=(
        jax.ShapeDtypeStruct((ELG,), f32),
        jax.ShapeDtypeStruct((ELG,), i32),
        jax.ShapeDtypeStruct((E,), f32),
    ),
    mesh=_mesh,
    scratch_types=[
        pltpu.VMEM((B1,), i32),
        pltpu.VMEM((B1,), i32),
        pltpu.VMEM((B1,), i32),
        pltpu.VMEM((B1,), f32),
        pltpu.VMEM((B1,), f32),
        pltpu.VMEM((B1,), i32),
        pltpu.VMEM((B1,), f32),
        pltpu.VMEM((DRS,), f32),
        pltpu.VMEM_SHARED((EHP,), f32),
        pltpu.SemaphoreType.DMA,
    ],
    compiler_params=_sc_params,
)
def _k1(slg_hbm, dlg_hbm, asrc_hbm, adst_hbm, h_hbm,
        exv_hbm, tnode_hbm, den_hbm,
        idx_s, idx_d, idxl, va, vb, vt, vex, zb, dacc, sem):
    c = lax.axis_index("c")
    s = lax.axis_index("s")

    def zfill(i, _):
        zb[pl.ds(i * 16, 16)] = jnp.zeros((16,), f32)
        return 0
    lax.fori_loop(0, DRS // 16, zfill, 0)
    pltpu.sync_copy(zb, dacc.at[pl.ds(s * DRS, DRS)])
    plsc.subcore_barrier()

    base = s * EPS1
    seg0 = c * EH

    def batch(bi, _):
        off = base + bi * B1
        pltpu.sync_copy(slg_hbm.at[pl.ds(off, B1)], idx_s)
        pltpu.sync_copy(dlg_hbm.at[pl.ds(off, B1)], idx_d)
        c1 = pltpu.async_copy(asrc_hbm.at[idx_s], va, sem)
        c2 = pltpu.async_copy(adst_hbm.at[idx_d], vb, sem)

        @pl.when(c == 0)
        def _():
            pltpu.async_copy(h_hbm.at[idx_d], vt, sem).wait()
        c1.wait()
        c2.wait()

        def cb(j, _):
            sl = pl.ds(j * 16, 16)
            a = va[sl] + vb[sl]
            a = jnp.where(a > 0.0, a, 0.2 * a)
            vex[sl] = jnp.exp(a)
            v = idx_d[sl] - seg0
            ok = (v >= 0) & (v < EH)
            idxl[sl] = jnp.where(ok, v, EH)
            return 0
        lax.fori_loop(0, B1 // 16, cb, 0)

        pltpu.sync_copy(vex, dacc.at[idxl], add=True)

        @pl.when(c == 0)
        def _():
            pltpu.sync_copy(vex, exv_hbm.at[pl.ds(off, B1)])
            pltpu.sync_copy(vt, tnode_hbm.at[pl.ds(off, B1)])
        return 0
    lax.fori_loop(0, EPS1 // B1, batch, 0)

    plsc.subcore_barrier()

    @pl.when(s < NS - 1)
    def _():
        pltpu.sync_copy(dacc.at[pl.ds(s * DRS, DRS)], zb)
        pltpu.sync_copy(zb, den_hbm.at[pl.ds(seg0 + s * DRS, DRS)])

    @pl.when(s == NS - 1)
    def _():
        pltpu.sync_copy(dacc.at[pl.ds(15 * DRS, 9760)], zb.at[pl.ds(0, 9760)])
        pltpu.sync_copy(zb.at[pl.ds(0, 9760)],
                        den_hbm.at[pl.ds(seg0 + 15 * DRS, 9760)])


B2 = 800
EPW2 = ELG // NW


@functools.partial(
    pl.kernel,
    out_type=jax.ShapeDtypeStruct((NC * 2 * N, CW), f32),
    mesh=_mesh,
    scratch_types=[
        pltpu.VMEM((B2,), i32),
        pltpu.VMEM((B2,), i32),
        pltpu.VMEM((B2,), i32),
        pltpu.VMEM((B2,), f32),
        pltpu.VMEM((B2,), f32),
        pltpu.VMEM((EPW2,), f32),
        pltpu.VMEM((B2, CW), f32),
        pltpu.VMEM((B2, CW), f32),
        pltpu.VMEM((SR, CW), f32),
        pltpu.VMEM_SHARED((N, CW), f32),
        pltpu.SemaphoreType.DMA,
    ],
    compiler_params=_sc_params,
)
def _k2(slg_hbm, dlg_hbm, exv_hbm, tnode_hbm, den_hbm, hga_hbm, hgb_hbm,
        nfep_hbm,
        idx_s, idx_d, vt, vex, vden, coefb, rows, rows_s, zb, dacc, sem):
    c = lax.axis_index("c")
    s = lax.axis_index("s")
    wid = s * NC + c
    base = wid * EPW2

    _fill_zero2d(zb)
    for q, tab in enumerate((hga_hbm, hgb_hbm)):
        _acc_zero_n(dacc, zb, s)
        plsc.subcore_barrier()

        def batch(bi, _):
            off = base + bi * B2
            loc = bi * B2
            pltpu.sync_copy(tnode_hbm.at[pl.ds(off, B2)], vt)
            pltpu.sync_copy(slg_hbm.at[pl.ds(off, B2)], idx_s)
            if q == 0:
                pltpu.sync_copy(dlg_hbm.at[pl.ds(off, B2)], idx_d)
                pltpu.sync_copy(exv_hbm.at[pl.ds(off, B2)], vex)
                c1 = pltpu.async_copy(den_hbm.at[idx_d], vden, sem)
                c2 = pltpu.async_copy(tab.at[idx_s], rows, sem)
                c1.wait()
                c2.wait()

                def coef(j, _):
                    sl = pl.ds(j * 16, 16)
                    coefb[pl.ds(loc + j * 16, 16)] = (
                        vex[sl] / (vden[sl] + 1e-16))
                    return 0
                lax.fori_loop(0, B2 // 16, coef, 0)
            else:
                pltpu.async_copy(tab.at[idx_s], rows, sem).wait()

            def scale(g, _):
                cf16 = coefb[pl.ds(loc + g * 16, 16)]
                for lane in range(16):
                    cf = cf16[lane]
                    r = g * 16 + lane
                    for jj in range(CW // 16):
                        sl = pl.ds(jj * 16, 16)
                        rows_s[r, sl] = rows[r, sl] * cf
                return 0
            lax.fori_loop(0, B2 // 16, scale, 0)

            pltpu.sync_copy(rows_s, dacc.at[vt], add=True)
            return 0
        lax.fori_loop(0, EPW2 // B2, batch, 0)

        plsc.subcore_barrier()
        _acc_readout_n(dacc, nfep_hbm, zb, (c * 2 + q) * N, s)
        plsc.subcore_barrier()
        _fill_zero2d(zb)


B3 = 2000
EPW3 = E // NW
DPS = ND // NS


def _make_k3(want_deg):
    outs = [jax.ShapeDtypeStruct((NC * 4 * N, CW), f32)]
    scratch = [
        pltpu.VMEM((B3,), i32),
        pltpu.VMEM((B3,), i32),
        pltpu.VMEM((B3, CW), f32),
        pltpu.VMEM((SR, CW), f32),
        pltpu.VMEM_SHARED((N, CW), f32),
        pltpu.SemaphoreType.DMA,
    ]
    if want_deg:
        outs.append(jax.ShapeDtypeStruct((NC * ND,), f32))
        scratch += [
            pltpu.VMEM((B3,), f32),
            pltpu.VMEM((DPS,), f32),
            pltpu.VMEM_SHARED((ND,), f32),
        ]

    def body(src_hbm, dst_hbm, t0_hbm, t1_hbm, t2_hbm, t3_hbm, *refs):
        if want_deg:
            (accp_hbm, degp_hbm,
             idx_s, idx_d, rows, zb, dacc, sem, vone, zb1, gacc) = refs
        else:
            (accp_hbm, idx_s, idx_d, rows, zb, dacc, sem) = refs
        c = lax.axis_index("c")
        s = lax.axis_index("s")
        wid = s * NC + c
        base = wid * EPW3

        _fill_zero2d(zb)
        if want_deg:
            def zfill1(i, _):
                zb1[pl.ds(i * 16, 16)] = jnp.zeros((16,), f32)
                return 0
            lax.fori_loop(0, DPS // 16, zfill1, 0)

            def onefill(i, _):
                vone[pl.ds(i * 16, 16)] = jnp.ones((16,), f32)
                return 0
            lax.fori_loop(0, B3 // 16, onefill, 0)
            pltpu.sync_copy(zb1, gacc.at[pl.ds(s * DPS, DPS)])

        for q, tab in enumerate((t0_hbm, t1_hbm, t2_hbm, t3_hbm)):
            _acc_zero_n(dacc, zb, s)
            plsc.subcore_barrier()

            def batch(bi, _):
                off = base + bi * B3
                pltpu.sync_copy(src_hbm.at[pl.ds(off, B3)], idx_s)
                pltpu.sync_copy(dst_hbm.at[pl.ds(off, B3)], idx_d)
                pltpu.async_copy(tab.at[idx_s], rows, sem).wait()
                pltpu.sync_copy(rows, dacc.at[idx_d], add=True)
                if want_deg and q == 0:
                    pltpu.sync_copy(vone, gacc.at[idx_d], add=True)
                return 0
            lax.fori_loop(0, EPW3 // B3, batch, 0)

            plsc.subcore_barrier()
            _acc_readout_n(dacc, accp_hbm, zb, (c * 4 + q) * N, s)
            plsc.subcore_barrier()
            _fill_zero2d(zb)

        if want_deg:
            pltpu.sync_copy(gacc.at[pl.ds(s * DPS, DPS)], zb1)
            pltpu.sync_copy(zb1, degp_hbm.at[pl.ds(c * ND + s * DPS, DPS)])

    return pl.kernel(body, out_type=tuple(outs) if want_deg else outs[0],
                     mesh=_mesh, scratch_types=scratch,
                     compiler_params=_sc_params)


_k3_deg = _make_k3(True)
_k3_nodeg = _make_k3(False)


BN = 2000

def _t23_body(x_ref, nfep_ref, accx_ref, degp_ref,
              wetn_ref, wegcn_ref, wn1s_ref, wn1n_ref,
              q0_ref, q1_ref, q2_ref, q3_ref):
    nfe = jnp.concatenate(
        [nfep_ref[0, 0] + nfep_ref[1, 0], nfep_ref[0, 1] + nfep_ref[1, 1]],
        axis=1)
    t = nfe @ wetn_ref[...]
    t = jnp.where(t > 0.0, t, 0.2 * t)
    er = t @ wegcn_ref[...]
    deg = jnp.maximum(degp_ref[0] + degp_ref[1], 1.0)
    inv = 1.0 / deg
    nx = jnp.concatenate(
        [accx_ref[0, q] + accx_ref[1, q] for q in range(4)], axis=1) * inv
    n1 = jnp.maximum(x_ref[...] @ wn1s_ref[...] + nx @ wn1n_ref[...], 0.0)
    q0_ref[...] = n1[:, 0:CW]
    q1_ref[...] = n1[:, CW:HID]
    q2_ref[...] = er[:, 0:CW]
    q3_ref[...] = er[:, CW:HID]


def _t23(x, nfep, accx, degp, wetn, wegcn, wn1s, wn1n):
    qspec = pl.BlockSpec((BN, CW), lambda i: (i, 0))
    return pl.pallas_call(
        _t23_body,
        grid=(N // BN,),
        in_specs=[
            pl.BlockSpec((BN, FIN), lambda i: (i, 0)),
            pl.BlockSpec((2, 2, BN, CW), lambda i: (0, 0, i, 0)),
            pl.BlockSpec((2, 4, BN, CW), lambda i: (0, 0, i, 0)),
            pl.BlockSpec((2, BN, 1), lambda i: (0, i, 0)),
            pl.BlockSpec((HID, HID), lambda i: (0, 0)),
            pl.BlockSpec((HID, HID), lambda i: (0, 0)),
            pl.BlockSpec((FIN, HID), lambda i: (0, 0)),
            pl.BlockSpec((FIN, HID), lambda i: (0, 0)),
        ],
        out_specs=[qspec, qspec, qspec, qspec],
        out_shape=[jax.ShapeDtypeStruct((N, CW), f32) for _ in range(4)],
    )(x, nfep, accx, degp, wetn, wegcn, wn1s, wn1n)


def _t4_body(q0_ref, q1_ref, q2_ref, q3_ref, acc2_ref, degp_ref,
             weas_ref, wean_ref, wn2s_ref, wn2n_ref, wq_ref, wk_ref, wo_ref,
             o_ref):
    n1 = jnp.concatenate([q0_ref[...], q1_ref[...]], axis=1)
    er = jnp.concatenate([q2_ref[...], q3_ref[...]], axis=1)
    deg = jnp.maximum(degp_ref[0] + degp_ref[1], 1.0)
    inv = 1.0 / deg
    neigh = jnp.concatenate(
        [acc2_ref[0, q] + acc2_ref[1, q] for q in range(4)], axis=1) * inv
    nn1 = neigh[:, 0:HID]
    ner = neigh[:, HID:FIN]
    aggr = er @ weas_ref[...] + ner @ wean_ref[...]
    nr = n1 @ wn2s_ref[...] + nn1 @ wn2n_ref[...]
    q = nr @ wq_ref[...]
    k1 = nr @ wk_ref[...]
    k2 = aggr @ wk_ref[...]
    s1 = jnp.sum(q * k1, axis=1) / 8.0
    s2 = jnp.sum(q * k2, axis=1) / 8.0
    m = jnp.maximum(s1, s2)
    e1 = jnp.exp(s1 - m)
    e2 = jnp.exp(s2 - m)
    zden = e1 + e2
    a1 = (e1 / zden)[:, None]
    a2 = (e2 / zden)[:, None]
    mix = a1 * nr + a2 * aggr
    lg = mix @ wo_ref[...]
    mm = jnp.max(lg, axis=1, keepdims=True)
    ls = mm + jnp.log(jnp.sum(jnp.exp(lg - mm), axis=1, keepdims=True))
    o_ref[...] = lg - ls


def _t4(q0, q1, q2, q3, acc2, degp, weas, wean, wn2s, wn2n, wq, wk, wo):
    qspec = pl.BlockSpec((BN, CW), lambda i: (i, 0))
    return pl.pallas_call(
        _t4_body,
        grid=(N // BN,),
        in_specs=[
            qspec, qspec, qspec, qspec,
            pl.BlockSpec((2, 4, BN, CW), lambda i: (0, 0, i, 0)),
            pl.BlockSpec((2, BN, 1), lambda i: (0, i, 0)),
            pl.BlockSpec((HID, HID), lambda i: (0, 0)),
            pl.BlockSpec((HID, HID), lambda i: (0, 0)),
            pl.BlockSpec((HID, HID), lambda i: (0, 0)),
            pl.BlockSpec((HID, HID), lambda i: (0, 0)),
            pl.BlockSpec((HID, HID), lambda i: (0, 0)),
            pl.BlockSpec((HID, HID), lambda i: (0, 0)),
            pl.BlockSpec((HID, OUT), lambda i: (0, 0)),
        ],
        out_specs=pl.BlockSpec((BN, OUT), lambda i: (i, 0)),
        out_shape=jax.ShapeDtypeStruct((N, OUT), f32),
    )(q0, q1, q2, q3, acc2, degp, weas, wean, wn2s, wn2n, wq, wk, wo)


def kernel(x, et, ea, H, raw_edge_index, lg_edge_index,
           W_in, b_in, W_gat, a_src, a_dst, W_etn, W_egcn,
           W_ea_self, W_ea_neigh, W_n1_self, W_n1_neigh,
           W_n2_self, W_n2_neigh, W_q, W_k, W_out):
    av2 = jnp.stack([a_src, a_dst])
    hga, hgb, sd = _t1(et, ea, W_in[:16], W_in[16:], b_in.reshape(1, HID),
                       W_gat, av2)
    asrc = sd[:, 0, :].reshape(E)
    adst = sd[:, 1, :].reshape(E)

    s_lg = lg_edge_index[0]
    d_lg = lg_edge_index[1]
    exv, tnode, den = _k1(s_lg, d_lg, asrc, adst, H)
    nfep = _k2(s_lg, d_lg, exv, tnode, den, hga, hgb).reshape(NC, 2, N, CW)

    e_src = raw_edge_index[0]
    e_dst = raw_edge_index[1]
    xq = [x[:, q * CW:(q + 1) * CW] for q in range(4)]
    accx, degp = _k3_deg(e_src, e_dst, *xq)
    accx = accx.reshape(NC, 4, N, CW)
    degp = degp.reshape(NC, ND)[:, :N].reshape(NC, N, 1)

    q0, q1, q2, q3 = _t23(x, nfep, accx, degp,
                          W_etn, W_egcn, W_n1_self, W_n1_neigh)
    acc2 = _k3_nodeg(e_src, e_dst, q0, q1, q2, q3).reshape(NC, 4, N, CW)
    return _t4(q0, q1, q2, q3, acc2, degp, W_ea_self, W_ea_neigh,
               W_n2_self, W_n2_neigh, W_q, W_k, W_out)


# speedup vs baseline: 1.0209x; 1.0209x over previous
import functools

import jax
import jax.numpy as jnp
from jax import lax
from jax.experimental import pallas as pl
from jax.experimental.pallas import tpu as pltpu
from jax.experimental.pallas import tpu_sc as plsc

N = 10000
E = 320000
ELG = 640000
HID = 64
FIN = 128
OUT = 40

NC = 2
NS = 16
NW = NC * NS
EH = E // NC
EHP = 160256
ND = 10240
CW = 32

_mesh = plsc.VectorSubcoreMesh(
    core_axis_name="c", subcore_axis_name="s", num_cores=NC, num_subcores=NS)
_sc_params = pltpu.CompilerParams(use_tc_tiling_on_sc=False)

f32 = jnp.float32
i32 = jnp.int32

SR = 320


def _fill_zero2d(stage):
    def zfill(i, _):
        for jj in range(CW // 16):
            stage[i, pl.ds(jj * 16, 16)] = jnp.zeros((16,), f32)
        return 0
    lax.fori_loop(0, SR, zfill, 0)


def _acc_zero_n(dacc, stage, s):
    @pl.when(s < NS - 1)
    def _():
        def cp(k, _):
            pltpu.sync_copy(stage, dacc.at[pl.ds(s * 640 + k * SR, SR)])
            return 0
        lax.fori_loop(0, 2, cp, 0)

    @pl.when(s == NS - 1)
    def _():
        pltpu.sync_copy(stage, dacc.at[pl.ds(9600, SR)])
        pltpu.sync_copy(stage.at[pl.ds(0, 80)], dacc.at[pl.ds(9920, 80)])


def _acc_readout_n(dacc, hbm_ref, stage, cbase, s):
    @pl.when(s < NS - 1)
    def _():
        def cp(k, _):
            r0 = s * 640 + k * SR
            pltpu.sync_copy(dacc.at[pl.ds(r0, SR)], stage)
            pltpu.sync_copy(stage, hbm_ref.at[pl.ds(cbase + r0, SR)])
            return 0
        lax.fori_loop(0, 2, cp, 0)

    @pl.when(s == NS - 1)
    def _():
        pltpu.sync_copy(dacc.at[pl.ds(9600, SR)], stage)
        pltpu.sync_copy(stage, hbm_ref.at[pl.ds(cbase + 9600, SR)])
        pltpu.sync_copy(dacc.at[pl.ds(9920, 80)], stage.at[pl.ds(0, 80)])
        pltpu.sync_copy(stage.at[pl.ds(0, 80)], hbm_ref.at[pl.ds(cbase + 9920, 80)])


BE = 2560

def _t1_body(et_ref, ea_ref, wa_ref, wb_ref, b_ref, wg_ref, av_ref,
             hga_ref, hgb_ref, sd_ref):
    z = et_ref[...] @ wa_ref[...] + ea_ref[...] @ wb_ref[...] + b_ref[...]
    h = jnp.maximum(z, 0.0)
    hg = h @ wg_ref[...]
    hga_ref[...] = hg[:, 0:CW]
    hgb_ref[...] = hg[:, CW:HID]
    s = lax.dot_general(av_ref[...], hg, (((1,), (1,)), ((), ())),
                        preferred_element_type=f32)
    sd_ref[0, :, :] = s


def _t1(et, ea, wa, wb, b, wg, av2):
    return pl.pallas_call(
        _t1_body,
        grid=(E // BE,),
        in_specs=[
            pl.BlockSpec((BE, 16), lambda i: (i, 0)),
            pl.BlockSpec((BE, 16), lambda i: (i, 0)),
            pl.BlockSpec((16, HID), lambda i: (0, 0)),
            pl.BlockSpec((16, HID), lambda i: (0, 0)),
            pl.BlockSpec((1, HID), lambda i: (0, 0)),
            pl.BlockSpec((HID, HID), lambda i: (0, 0)),
            pl.BlockSpec((2, HID), lambda i: (0, 0)),
        ],
        out_specs=[
            pl.BlockSpec((BE, CW), lambda i: (i, 0)),
            pl.BlockSpec((BE, CW), lambda i: (i, 0)),
            pl.BlockSpec((1, 2, BE), lambda i: (i, 0, 0)),
        ],
        out_shape=[
            jax.ShapeDtypeStruct((E, CW), f32),
            jax.ShapeDtypeStruct((E, CW), f32),
            jax.ShapeDtypeStruct((E // BE, 2, BE), f32),
        ],
    )(et, ea, wa, wb, b, wg, av2)


B1 = 2000
EPS1 = ELG // NS
DRS = EHP // NS


@functools.partial(
    pl.kernel,
    out_type=(
        jax.ShapeDtypeStruct((ELG,), f32),
        jax.ShapeDtypeStruct((ELG,), i32),
        jax.ShapeDtypeStruct((E,), f32),
    ),
    mesh=_mesh,
    scratch_types=[
        pltpu.VMEM((B1,), i32),
        pltpu.VMEM((B1,), i32),
        pltpu.VMEM((B1,), i32),
        pltpu.VMEM((B1,), f32),
        pltpu.VMEM((B1,), f32),
        pltpu.VMEM((B1,), i32),
        pltpu.VMEM((B1,), f32),
        pltpu.VMEM((DRS,), f32),
        pltpu.VMEM_SHARED((EHP,), f32),
        pltpu.SemaphoreType.DMA,
    ],
    compiler_params=_sc_params,
)
def _k1(slg_hbm, dlg_hbm, asrc_hbm, adst_hbm, h_hbm,
        exv_hbm, tnode_hbm, den_hbm,
        idx_s, idx_d, idxl, va, vb, vt, vex, zb, dacc, sem):
    c = lax.axis_index("c")
    s = lax.axis_index("s")

    def zfill(i, _):
        zb[pl.ds(i * 16, 16)] = jnp.zeros((16,), f32)
        return 0
    lax.fori_loop(0, DRS // 16, zfill, 0)
    pltpu.sync_copy(zb, dacc.at[pl.ds(s * DRS, DRS)])
    plsc.subcore_barrier()

    base = s * EPS1
    seg0 = c * EH

    def batch(bi, _):
        off = base + bi * B1
        pltpu.sync_copy(slg_hbm.at[pl.ds(off, B1)], idx_s)
        pltpu.sync_copy(dlg_hbm.at[pl.ds(off, B1)], idx_d)
        c1 = pltpu.async_copy(asrc_hbm.at[idx_s], va, sem)
        c2 = pltpu.async_copy(adst_hbm.at[idx_d], vb, sem)
        c3 = pltpu.async_copy(h_hbm.at[idx_d], vt, sem)
        c1.wait()
        c2.wait()
        c3.wait()

        def cb(j, _):
            sl = pl.ds(j * 16, 16)
            a = va[sl] + vb[sl]
            a = jnp.where(a > 0.0, a, 0.2 * a)
            vex[sl] = jnp.exp(a)
            v = idx_d[sl] - seg0
            ok = (v >= 0) & (v < EH)
            idxl[sl] = jnp.where(ok, v, EH)
            return 0
        lax.fori_loop(0, B1 // 16, cb, 0)

        pltpu.sync_copy(vex, dacc.at[idxl], add=True)

        @pl.when(c == 0)
        def _():
            pltpu.sync_copy(vex, exv_hbm.at[pl.ds(off, B1)])
            pltpu.sync_copy(vt, tnode_hbm.at[pl.ds(off, B1)])
        return 0
    lax.fori_loop(0, EPS1 // B1, batch, 0)

    plsc.subcore_barrier()

    @pl.when(s < NS - 1)
    def _():
        pltpu.sync_copy(dacc.at[pl.ds(s * DRS, DRS)], zb)
        pltpu.sync_copy(zb, den_hbm.at[pl.ds(seg0 + s * DRS, DRS)])

    @pl.when(s == NS - 1)
    def _():
        pltpu.sync_copy(dacc.at[pl.ds(15 * DRS, 9760)], zb.at[pl.ds(0, 9760)])
        pltpu.sync_copy(zb.at[pl.ds(0, 9760)],
                        den_hbm.at[pl.ds(seg0 + 15 * DRS, 9760)])


B2 = 800
EPW2 = ELG // NW


@functools.partial(
    pl.kernel,
    out_type=jax.ShapeDtypeStruct((NC * 2 * N, CW), f32),
    mesh=_mesh,
    scratch_types=[
        pltpu.VMEM((B2,), i32),
        pltpu.VMEM((B2,), i32),
        pltpu.VMEM((B2,), i32),
        pltpu.VMEM((B2,), f32),
        pltpu.VMEM((B2,), f32),
        pltpu.VMEM((EPW2,), f32),
        pltpu.VMEM((B2, CW), f32),
        pltpu.VMEM((B2, CW), f32),
        pltpu.VMEM((SR, CW), f32),
        pltpu.VMEM_SHARED((N, CW), f32),
        pltpu.SemaphoreType.DMA,
    ],
    compiler_params=_sc_params,
)
def _k2(slg_hbm, dlg_hbm, exv_hbm, tnode_hbm, den_hbm, hga_hbm, hgb_hbm,
        nfep_hbm,
        idx_s, idx_d, vt, vex, vden, coefb, rows, rows_s, zb, dacc, sem):
    c = lax.axis_index("c")
    s = lax.axis_index("s")
    wid = s * NC + c
    base = wid * EPW2

    _fill_zero2d(zb)
    for q, tab in enumerate((hga_hbm, hgb_hbm)):
        _acc_zero_n(dacc, zb, s)
        plsc.subcore_barrier()

        def batch(bi, _):
            off = base + bi * B2
            loc = bi * B2
            pltpu.sync_copy(tnode_hbm.at[pl.ds(off, B2)], vt)
            pltpu.sync_copy(slg_hbm.at[pl.ds(off, B2)], idx_s)
            if q == 0:
                pltpu.sync_copy(dlg_hbm.at[pl.ds(off, B2)], idx_d)
                pltpu.sync_copy(exv_hbm.at[pl.ds(off, B2)], vex)
                c1 = pltpu.async_copy(den_hbm.at[idx_d], vden, sem)
                c2 = pltpu.async_copy(tab.at[idx_s], rows, sem)
                c1.wait()
                c2.wait()

                def coef(j, _):
                    sl = pl.ds(j * 16, 16)
                    coefb[pl.ds(loc + j * 16, 16)] = (
                        vex[sl] / (vden[sl] + 1e-16))
                    return 0
                lax.fori_loop(0, B2 // 16, coef, 0)
            else:
                pltpu.async_copy(tab.at[idx_s], rows, sem).wait()

            def scale(g, _):
                cf16 = coefb[pl.ds(loc + g * 16, 16)]
                for lane in range(16):
                    cf = cf16[lane]
                    r = g * 16 + lane
                    for jj in range(CW // 16):
                        sl = pl.ds(jj * 16, 16)
                        rows_s[r, sl] = rows[r, sl] * cf
                return 0
            lax.fori_loop(0, B2 // 16, scale, 0)

            pltpu.sync_copy(rows_s, dacc.at[vt], add=True)
            return 0
        lax.fori_loop(0, EPW2 // B2, batch, 0)

        plsc.subcore_barrier()
        _acc_readout_n(dacc, nfep_hbm, zb, (c * 2 + q) * N, s)
        plsc.subcore_barrier()
        _fill_zero2d(zb)


B3 = 2000
EPW3 = E // NW
DPS = ND // NS


def _make_k3(want_deg):
    outs = [jax.ShapeDtypeStruct((NC * 4 * N, CW), f32)]
    scratch = [
        pltpu.VMEM((B3,), i32),
        pltpu.VMEM((B3,), i32),
        pltpu.VMEM((B3, CW), f32),
        pltpu.VMEM((SR, CW), f32),
        pltpu.VMEM_SHARED((N, CW), f32),
        pltpu.SemaphoreType.DMA,
    ]
    if want_deg:
        outs.append(jax.ShapeDtypeStruct((NC * ND,), f32))
        scratch += [
            pltpu.VMEM((B3,), f32),
            pltpu.VMEM((DPS,), f32),
            pltpu.VMEM_SHARED((ND,), f32),
        ]

    def body(src_hbm, dst_hbm, t0_hbm, t1_hbm, t2_hbm, t3_hbm, *refs):
        if want_deg:
            (accp_hbm, degp_hbm,
             idx_s, idx_d, rows, zb, dacc, sem, vone, zb1, gacc) = refs
        else:
            (accp_hbm, idx_s, idx_d, rows, zb, dacc, sem) = refs
        c = lax.axis_index("c")
        s = lax.axis_index("s")
        wid = s * NC + c
        base = wid * EPW3

        _fill_zero2d(zb)
        if want_deg:
            def zfill1(i, _):
                zb1[pl.ds(i * 16, 16)] = jnp.zeros((16,), f32)
                return 0
            lax.fori_loop(0, DPS // 16, zfill1, 0)

            def onefill(i, _):
                vone[pl.ds(i * 16, 16)] = jnp.ones((16,), f32)
                return 0
            lax.fori_loop(0, B3 // 16, onefill, 0)
            pltpu.sync_copy(zb1, gacc.at[pl.ds(s * DPS, DPS)])

        for q, tab in enumerate((t0_hbm, t1_hbm, t2_hbm, t3_hbm)):
            _acc_zero_n(dacc, zb, s)
            plsc.subcore_barrier()

            def batch(bi, _):
                off = base + bi * B3
                pltpu.sync_copy(src_hbm.at[pl.ds(off, B3)], idx_s)
                pltpu.sync_copy(dst_hbm.at[pl.ds(off, B3)], idx_d)
                pltpu.async_copy(tab.at[idx_s], rows, sem).wait()
                pltpu.sync_copy(rows, dacc.at[idx_d], add=True)
                if want_deg and q == 0:
                    pltpu.sync_copy(vone, gacc.at[idx_d], add=True)
                return 0
            lax.fori_loop(0, EPW3 // B3, batch, 0)

            plsc.subcore_barrier()
            _acc_readout_n(dacc, accp_hbm, zb, (c * 4 + q) * N, s)
            plsc.subcore_barrier()
            _fill_zero2d(zb)

        if want_deg:
            pltpu.sync_copy(gacc.at[pl.ds(s * DPS, DPS)], zb1)
            pltpu.sync_copy(zb1, degp_hbm.at[pl.ds(c * ND + s * DPS, DPS)])

    return pl.kernel(body, out_type=tuple(outs) if want_deg else outs[0],
                     mesh=_mesh, scratch_types=scratch,
                     compiler_params=_sc_params)


_k3_deg = _make_k3(True)
_k3_nodeg = _make_k3(False)


BN = 2000

def _t23_body(x_ref, nfep_ref, accx_ref, degp_ref,
              wetn_ref, wegcn_ref, wn1s_ref, wn1n_ref,
              q0_ref, q1_ref, q2_ref, q3_ref):
    nfe = jnp.concatenate(
        [nfep_ref[0, 0] + nfep_ref[1, 0], nfep_ref[0, 1] + nfep_ref[1, 1]],
        axis=1)
    t = nfe @ wetn_ref[...]
    t = jnp.where(t > 0.0, t, 0.2 * t)
    er = t @ wegcn_ref[...]
    deg = jnp.maximum(degp_ref[0] + degp_ref[1], 1.0)
    inv = 1.0 / deg
    nx = jnp.concatenate(
        [accx_ref[0, q] + accx_ref[1, q] for q in range(4)], axis=1) * inv
    n1 = jnp.maximum(x_ref[...] @ wn1s_ref[...] + nx @ wn1n_ref[...], 0.0)
    q0_ref[...] = n1[:, 0:CW]
    q1_ref[...] = n1[:, CW:HID]
    q2_ref[...] = er[:, 0:CW]
    q3_ref[...] = er[:, CW:HID]


def _t23(x, nfep, accx, degp, wetn, wegcn, wn1s, wn1n):
    qspec = pl.BlockSpec((BN, CW), lambda i: (i, 0))
    return pl.pallas_call(
        _t23_body,
        grid=(N // BN,),
        in_specs=[
            pl.BlockSpec((BN, FIN), lambda i: (i, 0)),
            pl.BlockSpec((2, 2, BN, CW), lambda i: (0, 0, i, 0)),
            pl.BlockSpec((2, 4, BN, CW), lambda i: (0, 0, i, 0)),
            pl.BlockSpec((2, BN, 1), lambda i: (0, i, 0)),
            pl.BlockSpec((HID, HID), lambda i: (0, 0)),
            pl.BlockSpec((HID, HID), lambda i: (0, 0)),
            pl.BlockSpec((FIN, HID), lambda i: (0, 0)),
            pl.BlockSpec((FIN, HID), lambda i: (0, 0)),
        ],
        out_specs=[qspec, qspec, qspec, qspec],
        out_shape=[jax.ShapeDtypeStruct((N, CW), f32) for _ in range(4)],
    )(x, nfep, accx, degp, wetn, wegcn, wn1s, wn1n)


def _t4_body(q0_ref, q1_ref, q2_ref, q3_ref, acc2_ref, degp_ref,
             weas_ref, wean_ref, wn2s_ref, wn2n_ref, wq_ref, wk_ref, wo_ref,
             o_ref):
    n1 = jnp.concatenate([q0_ref[...], q1_ref[...]], axis=1)
    er = jnp.concatenate([q2_ref[...], q3_ref[...]], axis=1)
    deg = jnp.maximum(degp_ref[0] + degp_ref[1], 1.0)
    inv = 1.0 / deg
    neigh = jnp.concatenate(
        [acc2_ref[0, q] + acc2_ref[1, q] for q in range(4)], axis=1) * inv
    nn1 = neigh[:, 0:HID]
    ner = neigh[:, HID:FIN]
    aggr = er @ weas_ref[...] + ner @ wean_ref[...]
    nr = n1 @ wn2s_ref[...] + nn1 @ wn2n_ref[...]
    q = nr @ wq_ref[...]
    k1 = nr @ wk_ref[...]
    k2 = aggr @ wk_ref[...]
    s1 = jnp.sum(q * k1, axis=1) / 8.0
    s2 = jnp.sum(q * k2, axis=1) / 8.0
    m = jnp.maximum(s1, s2)
    e1 = jnp.exp(s1 - m)
    e2 = jnp.exp(s2 - m)
    zden = e1 + e2
    a1 = (e1 / zden)[:, None]
    a2 = (e2 / zden)[:, None]
    mix = a1 * nr + a2 * aggr
    lg = mix @ wo_ref[...]
    mm = jnp.max(lg, axis=1, keepdims=True)
    ls = mm + jnp.log(jnp.sum(jnp.exp(lg - mm), axis=1, keepdims=True))
    o_ref[...] = lg - ls


def _t4(q0, q1, q2, q3, acc2, degp, weas, wean, wn2s, wn2n, wq, wk, wo):
    qspec = pl.BlockSpec((BN, CW), lambda i: (i, 0))
    return pl.pallas_call(
        _t4_body,
        grid=(N // BN,),
        in_specs=[
            qspec, qspec, qspec, qspec,
            pl.BlockSpec((2, 4, BN, CW), lambda i: (0, 0, i, 0)),
            pl.BlockSpec((2, BN, 1), lambda i: (0, i, 0)),
            pl.BlockSpec((HID, HID), lambda i: (0, 0)),
            pl.BlockSpec((HID, HID), lambda i: (0, 0)),
            pl.BlockSpec((HID, HID), lambda i: (0, 0)),
            pl.BlockSpec((HID, HID), lambda i: (0, 0)),
            pl.BlockSpec((HID, HID), lambda i: (0, 0)),
            pl.BlockSpec((HID, HID), lambda i: (0, 0)),
            pl.BlockSpec((HID, OUT), lambda i: (0, 0)),
        ],
        out_specs=pl.BlockSpec((BN, OUT), lambda i: (i, 0)),
        out_shape=jax.ShapeDtypeStruct((N, OUT), f32),
    )(q0, q1, q2, q3, acc2, degp, weas, wean, wn2s, wn2n, wq, wk, wo)


def kernel(x, et, ea, H, raw_edge_index, lg_edge_index,
           W_in, b_in, W_gat, a_src, a_dst, W_etn, W_egcn,
           W_ea_self, W_ea_neigh, W_n1_self, W_n1_neigh,
           W_n2_self, W_n2_neigh, W_q, W_k, W_out):
    av2 = jnp.stack([a_src, a_dst])
    hga, hgb, sd = _t1(et, ea, W_in[:16], W_in[16:], b_in.reshape(1, HID),
                       W_gat, av2)
    asrc = sd[:, 0, :].reshape(E)
    adst = sd[:, 1, :].reshape(E)

    s_lg = lg_edge_index[0]
    d_lg = lg_edge_index[1]
    exv, tnode, den = _k1(s_lg, d_lg, asrc, adst, H)
    nfep = _k2(s_lg, d_lg, exv, tnode, den, hga, hgb).reshape(NC, 2, N, CW)

    e_src = raw_edge_index[0]
    e_dst = raw_edge_index[1]
    xq = [x[:, q * CW:(q + 1) * CW] for q in range(4)]
    accx, degp = _k3_deg(e_src, e_dst, *xq)
    accx = accx.reshape(NC, 4, N, CW)
    degp = degp.reshape(NC, ND)[:, :N].reshape(NC, N, 1)

    q0, q1, q2, q3 = _t23(x, nfep, accx, degp,
                          W_etn, W_egcn, W_n1_self, W_n1_neigh)
    acc2 = _k3_nodeg(e_src, e_dst, q0, q1, q2, q3).reshape(NC, 4, N, CW)
    return _t4(q0, q1, q2, q3, acc2, degp, W_ea_self, W_ea_neigh,
               W_n2_self, W_n2_neigh, W_q, W_k, W_out)


# speedup vs baseline: 1.0215x; 1.0006x over previous
import functools

import jax
import jax.numpy as jnp
from jax import lax
from jax.experimental import pallas as pl
from jax.experimental.pallas import tpu as pltpu
from jax.experimental.pallas import tpu_sc as plsc

N = 10000
E = 320000
ELG = 640000
HID = 64
FIN = 128
OUT = 40

NC = 2
NS = 16
NW = NC * NS
EH = E // NC
EHP = 160256
ND = 10240
CW = 32

_mesh = plsc.VectorSubcoreMesh(
    core_axis_name="c", subcore_axis_name="s", num_cores=NC, num_subcores=NS)
_sc_params = pltpu.CompilerParams(use_tc_tiling_on_sc=False)

f32 = jnp.float32
i32 = jnp.int32

SR = 320


def _fill_zero2d(stage):
    def zfill(i, _):
        for jj in range(CW // 16):
            stage[i, pl.ds(jj * 16, 16)] = jnp.zeros((16,), f32)
        return 0
    lax.fori_loop(0, SR, zfill, 0)


def _acc_zero_n(dacc, stage, s):
    @pl.when(s < NS - 1)
    def _():
        def cp(k, _):
            pltpu.sync_copy(stage, dacc.at[pl.ds(s * 640 + k * SR, SR)])
            return 0
        lax.fori_loop(0, 2, cp, 0)

    @pl.when(s == NS - 1)
    def _():
        pltpu.sync_copy(stage, dacc.at[pl.ds(9600, SR)])
        pltpu.sync_copy(stage.at[pl.ds(0, 80)], dacc.at[pl.ds(9920, 80)])


def _acc_readout_n(dacc, hbm_ref, stage, cbase, s):
    @pl.when(s < NS - 1)
    def _():
        def cp(k, _):
            r0 = s * 640 + k * SR
            pltpu.sync_copy(dacc.at[pl.ds(r0, SR)], stage)
            pltpu.sync_copy(stage, hbm_ref.at[pl.ds(cbase + r0, SR)])
            return 0
        lax.fori_loop(0, 2, cp, 0)

    @pl.when(s == NS - 1)
    def _():
        pltpu.sync_copy(dacc.at[pl.ds(9600, SR)], stage)
        pltpu.sync_copy(stage, hbm_ref.at[pl.ds(cbase + 9600, SR)])
        pltpu.sync_copy(dacc.at[pl.ds(9920, 80)], stage.at[pl.ds(0, 80)])
        pltpu.sync_copy(stage.at[pl.ds(0, 80)], hbm_ref.at[pl.ds(cbase + 9920, 80)])


BE = 2560

def _t1_body(et_ref, ea_ref, wa_ref, wb_ref, b_ref, wg_ref, av_ref,
             hga_ref, hgb_ref, sd_ref):
    z = et_ref[...] @ wa_ref[...] + ea_ref[...] @ wb_ref[...] + b_ref[...]
    h = jnp.maximum(z, 0.0)
    hg = h @ wg_ref[...]
    hga_ref[...] = hg[:, 0:CW]
    hgb_ref[...] = hg[:, CW:HID]
    s = lax.dot_general(av_ref[...], hg, (((1,), (1,)), ((), ())),
                        preferred_element_type=f32)
    sd_ref[0, :, :] = s


def _t1(et, ea, wa, wb, b, wg, av2):
    return pl.pallas_call(
        _t1_body,
        grid=(E // BE,),
        in_specs=[
            pl.BlockSpec((BE, 16), lambda i: (i, 0)),
            pl.BlockSpec((BE, 16), lambda i: (i, 0)),
            pl.BlockSpec((16, HID), lambda i: (0, 0)),
            pl.BlockSpec((16, HID), lambda i: (0, 0)),
            pl.BlockSpec((1, HID), lambda i: (0, 0)),
            pl.BlockSpec((HID, HID), lambda i: (0, 0)),
            pl.BlockSpec((2, HID), lambda i: (0, 0)),
        ],
        out_specs=[
            pl.BlockSpec((BE, CW), lambda i: (i, 0)),
            pl.BlockSpec((BE, CW), lambda i: (i, 0)),
            pl.BlockSpec((1, 2, BE), lambda i: (i, 0, 0)),
        ],
        out_shape=[
            jax.ShapeDtypeStruct((E, CW), f32),
            jax.ShapeDtypeStruct((E, CW), f32),
            jax.ShapeDtypeStruct((E // BE, 2, BE), f32),
        ],
    )(et, ea, wa, wb, b, wg, av2)


B1 = 2000
EPS1 = ELG // NS
DRS = EHP // NS


@functools.partial(
    pl.kernel,
    out_type=(
        jax.ShapeDtypeStruct((ELG,), f32),
        jax.ShapeDtypeStruct((ELG,), i32),
        jax.ShapeDtypeStruct((E,), f32),
    ),
    mesh=_mesh,
    scratch_types=[
        pltpu.VMEM((B1,), i32),
        pltpu.VMEM((B1,), i32),
        pltpu.VMEM((B1,), i32),
        pltpu.VMEM((B1,), f32),
        pltpu.VMEM((B1,), f32),
        pltpu.VMEM((B1,), i32),
        pltpu.VMEM((B1,), f32),
        pltpu.VMEM((DRS,), f32),
        pltpu.VMEM_SHARED((EHP,), f32),
        pltpu.SemaphoreType.DMA,
    ],
    compiler_params=_sc_params,
)
def _k1(slg_hbm, dlg_hbm, asrc_hbm, adst_hbm, h_hbm,
        exv_hbm, tnode_hbm, den_hbm,
        idx_s, idx_d, idxl, va, vb, vt, vex, zb, dacc, sem):
    c = lax.axis_index("c")
    s = lax.axis_index("s")

    def zfill(i, _):
        zb[pl.ds(i * 16, 16)] = jnp.zeros((16,), f32)
        return 0
    lax.fori_loop(0, DRS // 16, zfill, 0)
    pltpu.sync_copy(zb, dacc.at[pl.ds(s * DRS, DRS)])
    plsc.subcore_barrier()

    base = s * EPS1
    seg0 = c * EH

    def batch(bi, _):
        off = base + bi * B1
        pltpu.sync_copy(slg_hbm.at[pl.ds(off, B1)], idx_s)
        pltpu.sync_copy(dlg_hbm.at[pl.ds(off, B1)], idx_d)
        c1 = pltpu.async_copy(asrc_hbm.at[idx_s], va, sem)
        c2 = pltpu.async_copy(adst_hbm.at[idx_d], vb, sem)
        c3 = pltpu.async_copy(h_hbm.at[idx_d], vt, sem)
        c1.wait()
        c2.wait()
        c3.wait()

        def cb(j, _):
            sl = pl.ds(j * 16, 16)
            a = va[sl] + vb[sl]
            a = jnp.where(a > 0.0, a, 0.2 * a)
            vex[sl] = jnp.exp(a)
            v = idx_d[sl] - seg0
            ok = (v >= 0) & (v < EH)
            idxl[sl] = jnp.where(ok, v, EH)
            return 0
        lax.fori_loop(0, B1 // 16, cb, 0)

        pltpu.sync_copy(vex, dacc.at[idxl], add=True)

        @pl.when(c == 0)
        def _():
            pltpu.sync_copy(vex, exv_hbm.at[pl.ds(off, B1)])
            pltpu.sync_copy(vt, tnode_hbm.at[pl.ds(off, B1)])
        return 0
    lax.fori_loop(0, EPS1 // B1, batch, 0)

    plsc.subcore_barrier()

    @pl.when(s < NS - 1)
    def _():
        pltpu.sync_copy(dacc.at[pl.ds(s * DRS, DRS)], zb)
        pltpu.sync_copy(zb, den_hbm.at[pl.ds(seg0 + s * DRS, DRS)])

    @pl.when(s == NS - 1)
    def _():
        pltpu.sync_copy(dacc.at[pl.ds(15 * DRS, 9760)], zb.at[pl.ds(0, 9760)])
        pltpu.sync_copy(zb.at[pl.ds(0, 9760)],
                        den_hbm.at[pl.ds(seg0 + 15 * DRS, 9760)])


B2 = 800
EPW2 = ELG // NW


@functools.partial(
    pl.kernel,
    out_type=jax.ShapeDtypeStruct((NC * 2 * N, CW), f32),
    mesh=_mesh,
    scratch_types=[
        pltpu.VMEM((B2,), i32),
        pltpu.VMEM((B2,), i32),
        pltpu.VMEM((B2,), i32),
        pltpu.VMEM((B2,), f32),
        pltpu.VMEM((B2,), f32),
        pltpu.VMEM((EPW2,), f32),
        pltpu.VMEM((B2, CW), f32),
        pltpu.VMEM((B2, CW), f32),
        pltpu.VMEM((SR, CW), f32),
        pltpu.VMEM_SHARED((N, CW), f32),
        pltpu.SemaphoreType.DMA,
    ],
    compiler_params=_sc_params,
)
def _k2(slg_hbm, dlg_hbm, exv_hbm, tnode_hbm, den_hbm, hga_hbm, hgb_hbm,
        nfep_hbm,
        idx_s, idx_d, vt, vex, vden, coefb, rows, rows_s, zb, dacc, sem):
    c = lax.axis_index("c")
    s = lax.axis_index("s")
    wid = s * NC + c
    base = wid * EPW2

    _fill_zero2d(zb)
    for q, tab in enumerate((hga_hbm, hgb_hbm)):
        _acc_zero_n(dacc, zb, s)
        plsc.subcore_barrier()

        def batch(bi, _):
            off = base + bi * B2
            loc = bi * B2
            pltpu.sync_copy(tnode_hbm.at[pl.ds(off, B2)], vt)
            pltpu.sync_copy(slg_hbm.at[pl.ds(off, B2)], idx_s)
            if q == 0:
                pltpu.sync_copy(dlg_hbm.at[pl.ds(off, B2)], idx_d)
                pltpu.sync_copy(exv_hbm.at[pl.ds(off, B2)], vex)
                c1 = pltpu.async_copy(den_hbm.at[idx_d], vden, sem)
                c2 = pltpu.async_copy(tab.at[idx_s], rows, sem)
                c1.wait()
                c2.wait()

                def coef(j, _):
                    sl = pl.ds(j * 16, 16)
                    coefb[pl.ds(loc + j * 16, 16)] = (
                        vex[sl] / (vden[sl] + 1e-16))
                    return 0
                lax.fori_loop(0, B2 // 16, coef, 0)
            else:
                pltpu.async_copy(tab.at[idx_s], rows, sem).wait()

            def scale(g, _):
                cf16 = coefb[pl.ds(loc + g * 16, 16)]
                for lane in range(16):
                    cf = cf16[lane]
                    r = g * 16 + lane
                    for jj in range(CW // 16):
                        sl = pl.ds(jj * 16, 16)
                        rows_s[r, sl] = rows[r, sl] * cf
                return 0
            lax.fori_loop(0, B2 // 16, scale, 0)

            pltpu.sync_copy(rows_s, dacc.at[vt], add=True)
            return 0
        lax.fori_loop(0, EPW2 // B2, batch, 0)

        plsc.subcore_barrier()
        _acc_readout_n(dacc, nfep_hbm, zb, (c * 2 + q) * N, s)
        plsc.subcore_barrier()
        _fill_zero2d(zb)


B3 = 2000
EPW3 = E // NW
DPS = ND // NS


def _make_k3(want_deg):
    outs = [jax.ShapeDtypeStruct((NC * 4 * N, CW), f32)]
    scratch = [
        pltpu.VMEM((B3,), i32),
        pltpu.VMEM((B3,), i32),
        pltpu.VMEM((B3, CW), f32),
        pltpu.VMEM((SR, CW), f32),
        pltpu.VMEM_SHARED((N, CW), f32),
        pltpu.SemaphoreType.DMA,
    ]
    if want_deg:
        outs.append(jax.ShapeDtypeStruct((NC * ND,), f32))
        scratch += [
            pltpu.VMEM((B3,), f32),
            pltpu.VMEM((DPS,), f32),
            pltpu.VMEM_SHARED((ND,), f32),
        ]

    def body(src_hbm, dst_hbm, t0_hbm, t1_hbm, t2_hbm, t3_hbm, *refs):
        if want_deg:
            (accp_hbm, degp_hbm,
             idx_s, idx_d, rows, zb, dacc, sem, vone, zb1, gacc) = refs
        else:
            (accp_hbm, idx_s, idx_d, rows, zb, dacc, sem) = refs
        c = lax.axis_index("c")
        s = lax.axis_index("s")
        wid = s * NC + c
        base = wid * EPW3

        _fill_zero2d(zb)
        if want_deg:
            def zfill1(i, _):
                zb1[pl.ds(i * 16, 16)] = jnp.zeros((16,), f32)
                return 0
            lax.fori_loop(0, DPS // 16, zfill1, 0)

            def onefill(i, _):
                vone[pl.ds(i * 16, 16)] = jnp.ones((16,), f32)
                return 0
            lax.fori_loop(0, B3 // 16, onefill, 0)
            pltpu.sync_copy(zb1, gacc.at[pl.ds(s * DPS, DPS)])

        for q, tab in enumerate((t0_hbm, t1_hbm, t2_hbm, t3_hbm)):
            _acc_zero_n(dacc, zb, s)
            plsc.subcore_barrier()

            def batch(bi, _):
                off = base + bi * B3
                pltpu.sync_copy(src_hbm.at[pl.ds(off, B3)], idx_s)
                pltpu.sync_copy(dst_hbm.at[pl.ds(off, B3)], idx_d)
                pltpu.async_copy(tab.at[idx_s], rows, sem).wait()
                pltpu.sync_copy(rows, dacc.at[idx_d], add=True)
                if want_deg and q == 0:
                    pltpu.sync_copy(vone, gacc.at[idx_d], add=True)
                return 0
            lax.fori_loop(0, EPW3 // B3, batch, 0)

            plsc.subcore_barrier()
            _acc_readout_n(dacc, accp_hbm, zb, (c * 4 + q) * N, s)
            plsc.subcore_barrier()
            _fill_zero2d(zb)

        if want_deg:
            pltpu.sync_copy(gacc.at[pl.ds(s * DPS, DPS)], zb1)
            pltpu.sync_copy(zb1, degp_hbm.at[pl.ds(c * ND + s * DPS, DPS)])

    return pl.kernel(body, out_type=tuple(outs) if want_deg else outs[0],
                     mesh=_mesh, scratch_types=scratch,
                     compiler_params=_sc_params)


_k3_deg = _make_k3(True)
_k3_nodeg = _make_k3(False)


BN = 2000

def _t23_body(x_ref, nfep_ref, accx_ref, degp_ref,
              wetn_ref, wegcn_ref, wn1s_ref, wn1n_ref,
              q0_ref, q1_ref, q2_ref, q3_ref):
    nfe = jnp.concatenate(
        [nfep_ref[0, 0] + nfep_ref[1, 0], nfep_ref[0, 1] + nfep_ref[1, 1]],
        axis=1)
    t = nfe @ wetn_ref[...]
    t = jnp.where(t > 0.0, t, 0.2 * t)
    er = t @ wegcn_ref[...]
    deg = jnp.maximum(degp_ref[0] + degp_ref[1], 1.0)
    inv = 1.0 / deg
    nx = jnp.concatenate(
        [accx_ref[0, q] + accx_ref[1, q] for q in range(4)], axis=1) * inv
    n1 = jnp.maximum(x_ref[...] @ wn1s_ref[...] + nx @ wn1n_ref[...], 0.0)
    q0_ref[...] = n1[:, 0:CW]
    q1_ref[...] = n1[:, CW:HID]
    q2_ref[...] = er[:, 0:CW]
    q3_ref[...] = er[:, CW:HID]


def _t23(x, nfep, accx, degp, wetn, wegcn, wn1s, wn1n):
    qspec = pl.BlockSpec((BN, CW), lambda i: (i, 0))
    return pl.pallas_call(
        _t23_body,
        grid=(N // BN,),
        in_specs=[
            pl.BlockSpec((BN, FIN), lambda i: (i, 0)),
            pl.BlockSpec((2, 2, BN, CW), lambda i: (0, 0, i, 0)),
            pl.BlockSpec((2, 4, BN, CW), lambda i: (0, 0, i, 0)),
            pl.BlockSpec((2, BN, 1), lambda i: (0, i, 0)),
            pl.BlockSpec((HID, HID), lambda i: (0, 0)),
            pl.BlockSpec((HID, HID), lambda i: (0, 0)),
            pl.BlockSpec((FIN, HID), lambda i: (0, 0)),
            pl.BlockSpec((FIN, HID), lambda i: (0, 0)),
        ],
        out_specs=[qspec, qspec, qspec, qspec],
        out_shape=[jax.ShapeDtypeStruct((N, CW), f32) for _ in range(4)],
    )(x, nfep, accx, degp, wetn, wegcn, wn1s, wn1n)


def _t4_body(q0_ref, q1_ref, q2_ref, q3_ref, acc2_ref, degp_ref,
             weas_ref, wean_ref, wn2s_ref, wn2n_ref, wq_ref, wk_ref, wo_ref,
             o_ref):
    n1 = jnp.concatenate([q0_ref[...], q1_ref[...]], axis=1)
    er = jnp.concatenate([q2_ref[...], q3_ref[...]], axis=1)
    deg = jnp.maximum(degp_ref[0] + degp_ref[1], 1.0)
    inv = 1.0 / deg
    neigh = jnp.concatenate(
        [acc2_ref[0, q] + acc2_ref[1, q] for q in range(4)], axis=1) * inv
    nn1 = neigh[:, 0:HID]
    ner = neigh[:, HID:FIN]
    aggr = er @ weas_ref[...] + ner @ wean_ref[...]
    nr = n1 @ wn2s_ref[...] + nn1 @ wn2n_ref[...]
    q = nr @ wq_ref[...]
    k1 = nr @ wk_ref[...]
    k2 = aggr @ wk_ref[...]
    s1 = jnp.sum(q * k1, axis=1) / 8.0
    s2 = jnp.sum(q * k2, axis=1) / 8.0
    m = jnp.maximum(s1, s2)
    e1 = jnp.exp(s1 - m)
    e2 = jnp.exp(s2 - m)
    zden = e1 + e2
    a1 = (e1 / zden)[:, None]
    a2 = (e2 / zden)[:, None]
    mix = a1 * nr + a2 * aggr
    lg = mix @ wo_ref[...]
    mm = jnp.max(lg, axis=1, keepdims=True)
    ls = mm + jnp.log(jnp.sum(jnp.exp(lg - mm), axis=1, keepdims=True))
    o_ref[...] = lg - ls


def _t4(q0, q1, q2, q3, acc2, degp, weas, wean, wn2s, wn2n, wq, wk, wo):
    qspec = pl.BlockSpec((BN, CW), lambda i: (i, 0))
    return pl.pallas_call(
        _t4_body,
        grid=(N // BN,),
        in_specs=[
            qspec, qspec, qspec, qspec,
            pl.BlockSpec((2, 4, BN, CW), lambda i: (0, 0, i, 0)),
            pl.BlockSpec((2, BN, 1), lambda i: (0, i, 0)),
            pl.BlockSpec((HID, HID), lambda i: (0, 0)),
            pl.BlockSpec((HID, HID), lambda i: (0, 0)),
            pl.BlockSpec((HID, HID), lambda i: (0, 0)),
            pl.BlockSpec((HID, HID), lambda i: (0, 0)),
            pl.BlockSpec((HID, HID), lambda i: (0, 0)),
            pl.BlockSpec((HID, HID), lambda i: (0, 0)),
            pl.BlockSpec((HID, OUT), lambda i: (0, 0)),
        ],
        out_specs=pl.BlockSpec((BN, OUT), lambda i: (i, 0)),
        out_shape=jax.ShapeDtypeStruct((N, OUT), f32),
    )(q0, q1, q2, q3, acc2, degp, weas, wean, wn2s, wn2n, wq, wk, wo)


def kernel(x, et, ea, H, raw_edge_index, lg_edge_index,
           W_in, b_in, W_gat, a_src, a_dst, W_etn, W_egcn,
           W_ea_self, W_ea_neigh, W_n1_self, W_n1_neigh,
           W_n2_self, W_n2_neigh, W_q, W_k, W_out):
    e_src = raw_edge_index[0]
    e_dst = raw_edge_index[1]
    xq = [x[:, q * CW:(q + 1) * CW] for q in range(4)]
    accx, degp = _k3_deg(e_src, e_dst, *xq)
    accx = accx.reshape(NC, 4, N, CW)
    degp = degp.reshape(NC, ND)[:, :N].reshape(NC, N, 1)

    av2 = jnp.stack([a_src, a_dst])
    hga, hgb, sd = _t1(et, ea, W_in[:16], W_in[16:], b_in.reshape(1, HID),
                       W_gat, av2)
    asrc = sd[:, 0, :].reshape(E)
    adst = sd[:, 1, :].reshape(E)

    s_lg = lg_edge_index[0]
    d_lg = lg_edge_index[1]
    exv, tnode, den = _k1(s_lg, d_lg, asrc, adst, H)
    nfep = _k2(s_lg, d_lg, exv, tnode, den, hga, hgb).reshape(NC, 2, N, CW)

    q0, q1, q2, q3 = _t23(x, nfep, accx, degp,
                          W_etn, W_egcn, W_n1_self, W_n1_neigh)
    acc2 = _k3_nodeg(e_src, e_dst, q0, q1, q2, q3).reshape(NC, 4, N, CW)
    return _t4(q0, q1, q2, q3, acc2, degp, W_ea_self, W_ea_neigh,
               W_n2_self, W_n2_neigh, W_q, W_k, W_out)
